# baseline (speedup 1.0000x reference)
import functools

import jax
import jax.numpy as jnp
from jax import lax
from jax.experimental import pallas as pl
from jax.experimental.pallas import tpu as pltpu
from jax.experimental.pallas import tpu_sc as plsc

N_BATCH = 2048
N_REL = 3
DEG = 16
FEAT_DIM = 512
EMBED_DIM = 512

NC = 2
NS = 16
NW = NC * NS
ROWS_PER_NODE = N_REL * DEG
IDX_PER_CHUNK = ROWS_PER_NODE
LANES = 16
COLS = FEAT_DIM // LANES

_SC_MESH = plsc.VectorSubcoreMesh(
    core_axis_name="c", subcore_axis_name="s", num_cores=NC, num_subcores=NS
)


def _make_sc(n_batch):
    b_per_w = n_batch // NW
    chunks = b_per_w

    @functools.partial(
        pl.kernel,
        out_type=jax.ShapeDtypeStruct((n_batch, FEAT_DIM), jnp.float32),
        mesh=_SC_MESH,
        scratch_types=[
            pltpu.VMEM((b_per_w,), jnp.int32),
            pltpu.VMEM((chunks, IDX_PER_CHUNK), jnp.int32),
            pltpu.VMEM((IDX_PER_CHUNK, FEAT_DIM), jnp.float32),
            pltpu.VMEM((IDX_PER_CHUNK, FEAT_DIM), jnp.float32),
            pltpu.VMEM((IDX_PER_CHUNK, FEAT_DIM), jnp.float32),
            pltpu.VMEM((b_per_w, FEAT_DIM), jnp.float32),
            pltpu.SemaphoreType.DMA,
            pltpu.SemaphoreType.DMA,
            pltpu.SemaphoreType.DMA,
            pltpu.SemaphoreType.DMA,
        ],
    )
    def sc_gather_agg(
        feats_hbm, self_idx_hbm, idx_hbm, agg_hbm, sidx_v, idx_v,
        buf0, buf1, buf2, out_v, ssem, sem0, sem1, sem2
    ):
        wid = lax.axis_index("s") * NC + lax.axis_index("c")
        pltpu.sync_copy(self_idx_hbm.at[pl.ds(wid * b_per_w, b_per_w)], sidx_v)
        pltpu.sync_copy(idx_hbm.at[wid], idx_v)
        self_cp = pltpu.async_copy(feats_hbm.at[sidx_v], out_v, ssem)

        inv_deg = jnp.float32(1.0 / DEG)

        def _start(ci, buf, sem):
            pltpu.async_copy(feats_hbm.at[idx_v.at[ci]], buf, sem)

        def _wait(ci, buf, sem):
            pltpu.make_async_copy(feats_hbm.at[idx_v.at[ci]], buf, sem).wait()

        def _accum(node, buf):
            def acc_body(k, acc):
                r = 2 * k
                return tuple(
                    acc[c]
                    + (
                        buf[r, pl.ds(c * LANES, LANES)]
                        + buf[r + 1, pl.ds(c * LANES, LANES)]
                    )
                    for c in range(COLS)
                )

            acc0 = tuple(
                jnp.zeros((LANES,), jnp.float32) for _ in range(COLS)
            )
            acc = lax.fori_loop(0, ROWS_PER_NODE // 2, acc_body, acc0)
            for c in range(COLS):
                sl = pl.ds(c * LANES, LANES)
                out_v[node, sl] = out_v[node, sl] + acc[c] * inv_deg

        ring = ((buf1, sem1), (buf2, sem2), (buf0, sem0))

        _start(0, buf0, sem0)
        _start(1, buf1, sem1)
        _start(2, buf2, sem2)
        self_cp.wait()
        _wait(0, buf0, sem0)
        _accum(0, buf0)
        _start(3, buf0, sem0)

        def tri_body(i, _):
            c = 3 * i
            for off, (b, s) in enumerate(ring, start=1):
                ci = c + off
                _wait(ci, b, s)
                _accum(ci, b)

                @pl.when(ci + 3 < chunks)
                def _prefetch():
                    _start(ci + 3, b, s)

            return _

        ntri = (chunks - 1) // 3
        lax.fori_loop(0, ntri, tri_body, None)
        for r in range((chunks - 1) % 3):
            ci = 1 + 3 * ntri + r
            b, s = ring[r]
            _wait(ci, b, s)
            _accum(ci, b)
        pltpu.sync_copy(out_v, agg_hbm.at[pl.ds(wid * b_per_w, b_per_w)])

    return sc_gather_agg


def _mm_body(agg_ref, w_ref, o_ref):
    o_ref[...] = jnp.maximum(
        jnp.dot(agg_ref[...], w_ref[...], preferred_element_type=jnp.float32)
        * 0.25,
        0.0,
    )


def _make_mm(n_batch, grid):
    return pl.pallas_call(
        _mm_body,
        out_shape=jax.ShapeDtypeStruct((n_batch, EMBED_DIM), jnp.float32),
        grid=(grid,),
        in_specs=[
            pl.BlockSpec((n_batch // grid, FEAT_DIM), lambda i: (i, 0)),
            pl.BlockSpec((FEAT_DIM, EMBED_DIM), lambda i: (0, 0)),
        ],
        out_specs=pl.BlockSpec((n_batch // grid, EMBED_DIM), lambda i: (i, 0)),
    )


_sc_full = _make_sc(N_BATCH)
_mm_full = _make_mm(N_BATCH, 2)


@jax.jit
def kernel(features, weight, nodes, neigh_idx):
    nodes = nodes.astype(jnp.int32)
    neigh_idx = neigh_idx.astype(jnp.int32)
    idx_all = neigh_idx.transpose(1, 0, 2).reshape(
        NW, N_BATCH // NW, IDX_PER_CHUNK
    )
    agg = _sc_full(features, nodes, idx_all)
    return _mm_full(agg, weight)


# speedup vs baseline: 1.0108x; 1.0108x over previous
import functools

import jax
import jax.numpy as jnp
from jax import lax
from jax.experimental import pallas as pl
from jax.experimental.pallas import tpu as pltpu
from jax.experimental.pallas import tpu_sc as plsc

N_BATCH = 2048
N_REL = 3
DEG = 16
FEAT_DIM = 512
EMBED_DIM = 512

NC = 2
NS = 16
NW = NC * NS
ROWS_PER_NODE = N_REL * DEG
IDX_PER_CHUNK = ROWS_PER_NODE
LANES = 16
COLS = FEAT_DIM // LANES

_SC_MESH = plsc.VectorSubcoreMesh(
    core_axis_name="c", subcore_axis_name="s", num_cores=NC, num_subcores=NS
)


def _make_sc(n_batch):
    b_per_w = n_batch // NW
    chunks = b_per_w

    @functools.partial(
        pl.kernel,
        out_type=jax.ShapeDtypeStruct((n_batch, FEAT_DIM), jnp.float32),
        mesh=_SC_MESH,
        scratch_types=[
            pltpu.VMEM((b_per_w,), jnp.int32),
            pltpu.VMEM((chunks, IDX_PER_CHUNK), jnp.int32),
            pltpu.VMEM((IDX_PER_CHUNK, FEAT_DIM), jnp.float32),
            pltpu.VMEM((IDX_PER_CHUNK, FEAT_DIM), jnp.float32),
            pltpu.VMEM((IDX_PER_CHUNK, FEAT_DIM), jnp.float32),
            pltpu.VMEM((b_per_w, FEAT_DIM), jnp.float32),
            pltpu.SemaphoreType.DMA,
            pltpu.SemaphoreType.DMA,
            pltpu.SemaphoreType.DMA,
            pltpu.SemaphoreType.DMA,
        ],
    )
    def sc_gather_agg(
        feats_hbm, self_idx_hbm, idx_hbm, agg_hbm, sidx_v, idx_v,
        buf0, buf1, buf2, out_v, ssem, sem0, sem1, sem2
    ):
        wid = lax.axis_index("s") * NC + lax.axis_index("c")
        pltpu.sync_copy(self_idx_hbm.at[pl.ds(wid * b_per_w, b_per_w)], sidx_v)
        pltpu.sync_copy(idx_hbm.at[wid], idx_v)
        self_cp = pltpu.async_copy(feats_hbm.at[sidx_v], out_v, ssem)

        inv_deg = jnp.float32(1.0 / DEG)

        def _start(ci, buf, sem):
            pltpu.async_copy(feats_hbm.at[idx_v.at[ci]], buf, sem)

        def _wait(ci, buf, sem):
            pltpu.make_async_copy(feats_hbm.at[idx_v.at[ci]], buf, sem).wait()

        def _accum(node, buf):
            def acc_body(k, acc):
                return tuple(
                    acc[c] + buf[k, pl.ds(c * LANES, LANES)]
                    for c in range(COLS)
                )

            acc0 = tuple(
                jnp.zeros((LANES,), jnp.float32) for _ in range(COLS)
            )
            acc = lax.fori_loop(0, ROWS_PER_NODE, acc_body, acc0)
            for c in range(COLS):
                sl = pl.ds(c * LANES, LANES)
                out_v[node, sl] = out_v[node, sl] + acc[c] * inv_deg

        ring = ((buf1, sem1), (buf2, sem2), (buf0, sem0))

        _start(0, buf0, sem0)
        _start(1, buf1, sem1)
        _start(2, buf2, sem2)
        self_cp.wait()
        _wait(0, buf0, sem0)
        _accum(0, buf0)
        _start(3, buf0, sem0)

        def tri_body(i, _):
            c = 3 * i
            for off, (b, s) in enumerate(ring, start=1):
                ci = c + off
                _wait(ci, b, s)
                _accum(ci, b)

                @pl.when(ci + 3 < chunks)
                def _prefetch():
                    _start(ci + 3, b, s)

            return _

        ntri = (chunks - 1) // 3
        lax.fori_loop(0, ntri, tri_body, None)
        for r in range((chunks - 1) % 3):
            ci = 1 + 3 * ntri + r
            b, s = ring[r]
            _wait(ci, b, s)
            _accum(ci, b)
        pltpu.sync_copy(out_v, agg_hbm.at[pl.ds(wid * b_per_w, b_per_w)])

    return sc_gather_agg


def _mm_body(agg_ref, w_ref, o_ref):
    o_ref[...] = jnp.maximum(
        jnp.dot(agg_ref[...], w_ref[...], preferred_element_type=jnp.float32)
        * 0.25,
        0.0,
    )


def _make_mm(n_batch, grid):
    return pl.pallas_call(
        _mm_body,
        out_shape=jax.ShapeDtypeStruct((n_batch, EMBED_DIM), jnp.float32),
        grid=(grid,),
        in_specs=[
            pl.BlockSpec((n_batch // grid, FEAT_DIM), lambda i: (i, 0)),
            pl.BlockSpec((FEAT_DIM, EMBED_DIM), lambda i: (0, 0)),
        ],
        out_specs=pl.BlockSpec((n_batch // grid, EMBED_DIM), lambda i: (i, 0)),
    )


_sc_full = _make_sc(N_BATCH)
_mm_full = _make_mm(N_BATCH, 2)


@jax.jit
def kernel(features, weight, nodes, neigh_idx):
    nodes = nodes.astype(jnp.int32)
    neigh_idx = neigh_idx.astype(jnp.int32)
    idx_all = neigh_idx.transpose(1, 0, 2).reshape(
        NW, N_BATCH // NW, IDX_PER_CHUNK
    )
    agg = _sc_full(features, nodes, idx_all)
    return _mm_full(agg, weight)


# speedup vs baseline: 1.0141x; 1.0033x over previous
import functools

import jax
import jax.numpy as jnp
from jax import lax
from jax.experimental import pallas as pl
from jax.experimental.pallas import tpu as pltpu
from jax.experimental.pallas import tpu_sc as plsc

N_BATCH = 2048
N_REL = 3
DEG = 16
FEAT_DIM = 512
EMBED_DIM = 512

NC = 2
NS = 16
NW = NC * NS
ROWS_PER_NODE = N_REL * DEG
IDX_PER_CHUNK = ROWS_PER_NODE
LANES = 16
COLS = FEAT_DIM // LANES

_SC_MESH = plsc.VectorSubcoreMesh(
    core_axis_name="c", subcore_axis_name="s", num_cores=NC, num_subcores=NS
)


def _make_sc(n_batch):
    b_per_w = n_batch // NW
    chunks = b_per_w

    @functools.partial(
        pl.kernel,
        out_type=jax.ShapeDtypeStruct((n_batch, FEAT_DIM), jnp.float32),
        mesh=_SC_MESH,
        scratch_types=[
            pltpu.VMEM((b_per_w,), jnp.int32),
            pltpu.VMEM((chunks, IDX_PER_CHUNK), jnp.int32),
            pltpu.VMEM((IDX_PER_CHUNK, FEAT_DIM), jnp.float32),
            pltpu.VMEM((IDX_PER_CHUNK, FEAT_DIM), jnp.float32),
            pltpu.VMEM((IDX_PER_CHUNK, FEAT_DIM), jnp.float32),
            pltpu.VMEM((b_per_w, FEAT_DIM), jnp.float32),
            pltpu.SemaphoreType.DMA,
            pltpu.SemaphoreType.DMA,
            pltpu.SemaphoreType.DMA,
            pltpu.SemaphoreType.DMA,
        ],
    )
    def sc_gather_agg(
        feats_hbm, self_idx_hbm, idx_hbm, agg_hbm, sidx_v, idx_v,
        buf0, buf1, buf2, out_v, ssem, sem0, sem1, sem2
    ):
        wid = lax.axis_index("s") * NC + lax.axis_index("c")
        pltpu.sync_copy(self_idx_hbm.at[pl.ds(wid * b_per_w, b_per_w)], sidx_v)
        pltpu.sync_copy(idx_hbm.at[wid], idx_v)
        self_cp = pltpu.async_copy(feats_hbm.at[sidx_v], out_v, ssem)

        inv_deg = jnp.float32(1.0 / DEG)

        def _start(ci, buf, sem):
            pltpu.async_copy(feats_hbm.at[idx_v.at[ci]], buf, sem)

        def _wait(ci, buf, sem):
            pltpu.make_async_copy(feats_hbm.at[idx_v.at[ci]], buf, sem).wait()

        def _accum(node, buf):
            def acc_body(k, acc):
                return tuple(
                    acc[c] + buf[k, pl.ds(c * LANES, LANES)]
                    for c in range(COLS)
                )

            acc0 = tuple(
                jnp.zeros((LANES,), jnp.float32) for _ in range(COLS)
            )
            acc = lax.fori_loop(0, ROWS_PER_NODE, acc_body, acc0)
            for c in range(COLS):
                sl = pl.ds(c * LANES, LANES)
                out_v[node, sl] = out_v[node, sl] + acc[c] * inv_deg

        ring = ((buf1, sem1), (buf2, sem2), (buf0, sem0))

        _start(0, buf0, sem0)
        _start(1, buf1, sem1)
        _start(2, buf2, sem2)
        self_cp.wait()
        _wait(0, buf0, sem0)
        _accum(0, buf0)
        _start(3, buf0, sem0)

        def tri_body(i, _):
            c = 3 * i
            for off, (b, s) in enumerate(ring, start=1):
                ci = c + off
                _wait(ci, b, s)
                _accum(ci, b)

                @pl.when(ci + 3 < chunks)
                def _prefetch():
                    _start(ci + 3, b, s)

            return _

        ntri = (chunks - 1) // 3
        lax.fori_loop(0, ntri, tri_body, None)
        for r in range((chunks - 1) % 3):
            ci = 1 + 3 * ntri + r
            b, s = ring[r]
            _wait(ci, b, s)
            _accum(ci, b)
        pltpu.sync_copy(out_v, agg_hbm.at[pl.ds(wid * b_per_w, b_per_w)])

    return sc_gather_agg


def _mm_body(agg_ref, w_ref, o_ref):
    o_ref[...] = jnp.maximum(
        jnp.dot(
            agg_ref[...].astype(jnp.bfloat16),
            w_ref[...].astype(jnp.bfloat16),
            preferred_element_type=jnp.float32,
        )
        * 0.25,
        0.0,
    )


def _make_mm(n_batch, grid):
    return pl.pallas_call(
        _mm_body,
        out_shape=jax.ShapeDtypeStruct((n_batch, EMBED_DIM), jnp.float32),
        grid=(grid,),
        in_specs=[
            pl.BlockSpec((n_batch // grid, FEAT_DIM), lambda i: (i, 0)),
            pl.BlockSpec((FEAT_DIM, EMBED_DIM), lambda i: (0, 0)),
        ],
        out_specs=pl.BlockSpec((n_batch // grid, EMBED_DIM), lambda i: (i, 0)),
    )


_sc_full = _make_sc(N_BATCH)
_mm_full = _make_mm(N_BATCH, 2)


@jax.jit
def kernel(features, weight, nodes, neigh_idx):
    nodes = nodes.astype(jnp.int32)
    neigh_idx = neigh_idx.astype(jnp.int32)
    idx_all = neigh_idx.transpose(1, 0, 2).reshape(
        NW, N_BATCH // NW, IDX_PER_CHUNK
    )
    agg = _sc_full(features, nodes, idx_all)
    return _mm_full(agg, weight)
